# baseline (speedup 1.0000x reference)
import jax
import jax.numpy as jnp
from jax.experimental import pallas as pl
from jax.experimental.pallas import tpu as pltpu

_PATCH = 8
_LANE = 128


def _ln(x, g, b, eps=1e-6):
    mu = jnp.mean(x, axis=-1, keepdims=True)
    var = jnp.mean((x - mu) ** 2, axis=-1, keepdims=True)
    return (x - mu) * jax.lax.rsqrt(var + eps) * g + b


def _bf(x):
    return x.astype(jnp.bfloat16)


def _fused_kernel(p4_ref,
                  w4, b_emb, ln1_g, ln1_b, qkv_w, qkv_b, proj_w, proj_b,
                  ln2_g, ln2_b, mlp1_w, mlp1_b, mlp2_w, mlp2_b,
                  neck_w, neck_b, w1f_eff, wimg, b1_eff, w2, b2,
                  out_ref, *, S, h, w, H, W, heads, dh):
    f32 = jnp.float32

    p = p4_ref[...]
    x = jnp.dot(p, w4[...], preferred_element_type=f32) + b_emb[...]

    hn = _ln(x, ln1_g[...], ln1_b[...])
    qkv = jnp.dot(_bf(hn), qkv_w[...], preferred_element_type=f32) + qkv_b[...]

    embed = hn.shape[-1]
    scale = 1.0 / (dh ** 0.5)
    head_outs = []
    for hd in range(heads):
        lo = hd * dh
        qh = qkv[:, lo:lo + dh] * scale
        kh = qkv[:, embed + lo:embed + lo + dh]
        vh = qkv[:, 2 * embed + lo:2 * embed + lo + dh]
        s = jax.lax.dot_general(_bf(qh), _bf(kh), (((1,), (1,)), ((), ())),
                                preferred_element_type=f32)
        m = jnp.max(s, axis=-1, keepdims=True)
        e = jnp.exp(s - m)
        e = e * pl.reciprocal(jnp.sum(e, axis=-1, keepdims=True), approx=True)
        head_outs.append(jnp.dot(_bf(e), _bf(vh), preferred_element_type=f32))
    attn = jnp.concatenate(head_outs, axis=-1)

    x = x + jnp.dot(_bf(attn), proj_w[...], preferred_element_type=f32) + proj_b[...]

    hn = _ln(x, ln2_g[...], ln2_b[...])
    mlp = jnp.dot(_bf(hn), mlp1_w[...], preferred_element_type=f32) + mlp1_b[...]
    mlp = jnp.maximum(mlp, 0.0)
    x = x + jnp.dot(_bf(mlp), mlp2_w[...], preferred_element_type=f32) + mlp2_b[...]

    feas = jnp.dot(_bf(x), neck_w[...], preferred_element_type=f32) + neck_b[...]

    d = jnp.dot(_bf(feas), w1f_eff[...], preferred_element_type=f32)
    d = d + jnp.dot(p, wimg[...], preferred_element_type=f32)
    d = jnp.maximum(d + b1_eff[...], 0.0)
    a = jnp.dot(_bf(d), w2[...], preferred_element_type=f32) + b2[...]
    alpha = jax.nn.sigmoid(a[:, 0:1])

    av = jnp.broadcast_to(alpha, (S, _LANE))
    t_i = jax.lax.broadcasted_iota(jnp.int32, (S, _LANE), 0)
    l_i = jax.lax.broadcasted_iota(jnp.int32, (S, _LANE), 1)
    asel = av * (t_i % w == l_i).astype(f32)
    r_i = jax.lax.broadcasted_iota(jnp.int32, (h, S), 0)
    r_t = jax.lax.broadcasted_iota(jnp.int32, (h, S), 1)
    sm = (r_t // w == r_i).astype(f32)
    g16 = jnp.dot(sm, asel, preferred_element_type=f32)
    u_r = jax.lax.broadcasted_iota(jnp.int32, (H, h), 0)
    u_c = jax.lax.broadcasted_iota(jnp.int32, (H, h), 1)
    rrow = (u_r // _PATCH == u_c).astype(f32)
    pv = jnp.dot(rrow, g16, preferred_element_type=f32)
    c_r = jax.lax.broadcasted_iota(jnp.int32, (_LANE, W), 0)
    c_c = jax.lax.broadcasted_iota(jnp.int32, (_LANE, W), 1)
    rcol = (c_r == c_c // _PATCH).astype(f32)
    pred = jnp.dot(pv, rcol, preferred_element_type=f32)

    out_ref[...] = pred.reshape(1, 1, H, W)


def _full_spec(shape):
    nd = len(shape)
    return pl.BlockSpec(tuple(shape), lambda *_: (0,) * nd)


def kernel(patch_w, patch_b, guide_w, guide_b, ln1_g, ln1_b, qkv_w, qkv_b,
           proj_w, proj_b, ln2_g, ln2_b, mlp1_w, mlp1_b, mlp2_w, mlp2_b,
           neck_w, neck_b, maskhead_w, maskhead_b, m2m1_feas_w, m2m1_img_w,
           m2m1_mask_w, m2m1_b, m2m2_w, m2m2_b, image, guidance):
    f32 = jnp.float32
    B, Cin, H, W = image.shape
    h, w = H // _PATCH, W // _PATCH
    S = h * w
    M = B * S
    embed = patch_w.shape[1]
    heads = 4
    dh = embed // heads
    npix = _PATCH * _PATCH

    pw = patch_w.astype(f32).reshape(npix, Cin, embed)
    pw = pw.transpose(1, 0, 2).reshape(Cin * npix, embed)
    gw = jnp.broadcast_to(guide_w.astype(f32) / npix, (npix, embed))
    w4 = jnp.concatenate([pw, gw], axis=0).astype(jnp.bfloat16)
    b_emb = patch_b + guide_b

    w1f_eff = (m2m1_feas_w.astype(f32)
               + maskhead_w.astype(f32) @ m2m1_mask_w.astype(f32))
    w1f_eff = w1f_eff.astype(jnp.bfloat16)
    b1_eff = m2m1_b + maskhead_b @ m2m1_mask_w.astype(f32)

    dec_h = m2m1_img_w.shape[1]
    wi = jnp.broadcast_to(m2m1_img_w.astype(f32)[:, None] / npix,
                          (Cin, npix, dec_h)).reshape(Cin * npix, dec_h)
    wimg = jnp.concatenate([wi, jnp.zeros((npix, dec_h), f32)], axis=0)
    wimg = wimg.astype(jnp.bfloat16)

    weights = [w4, b_emb, ln1_g, ln1_b, qkv_w, qkv_b, proj_w, proj_b,
               ln2_g, ln2_b, mlp1_w, mlp1_b, mlp2_w, mlp2_b,
               neck_w, neck_b, w1f_eff, wimg, b1_eff, m2m2_w, m2m2_b]
    w_specs = [_full_spec(x.shape) for x in weights]

    import functools
    body = functools.partial(_fused_kernel, S=S, h=h, w=w, H=H, W=W,
                             heads=heads, dh=dh)
    xcat = jnp.concatenate([image, guidance], axis=1)
    p4 = xcat.reshape(B, Cin + 1, h, _PATCH, w, _PATCH)
    p4 = p4.transpose(0, 2, 4, 1, 3, 5).reshape(M, npix * (Cin + 1))
    p4 = p4.astype(jnp.bfloat16)

    pred = pl.pallas_call(
        body,
        out_shape=jax.ShapeDtypeStruct((B, 1, H, W), f32),
        grid=(B,),
        in_specs=[pl.BlockSpec((S, npix * (Cin + 1)), lambda b: (b, 0))] + w_specs,
        out_specs=pl.BlockSpec((1, 1, H, W), lambda b: (b, 0, 0, 0)),
        compiler_params=pltpu.CompilerParams(
            dimension_semantics=("parallel",)),
    )(p4, *weights)
    return pred


# speedup vs baseline: 1.3009x; 1.3009x over previous
import functools

import jax
import jax.numpy as jnp
from jax.experimental import pallas as pl
from jax.experimental.pallas import tpu as pltpu

_PATCH = 8
_LANE = 128


def _ln(x, g, b, eps=1e-6):
    mu = jnp.mean(x, axis=-1, keepdims=True)
    var = jnp.mean((x - mu) ** 2, axis=-1, keepdims=True)
    return (x - mu) * jax.lax.rsqrt(var + eps) * g + b


def _bf(x):
    return x.astype(jnp.bfloat16)


def _upsample(alpha, h, w, H, W):
    f32 = jnp.float32
    S = h * w
    av = jnp.broadcast_to(alpha, (S, _LANE))
    t_i = jax.lax.broadcasted_iota(jnp.int32, (S, _LANE), 0)
    l_i = jax.lax.broadcasted_iota(jnp.int32, (S, _LANE), 1)
    asel = av * (t_i % w == l_i).astype(f32)
    r_i = jax.lax.broadcasted_iota(jnp.int32, (h, S), 0)
    r_t = jax.lax.broadcasted_iota(jnp.int32, (h, S), 1)
    sm = (r_t // w == r_i).astype(f32)
    g16 = jnp.dot(sm, asel, preferred_element_type=f32)
    u_r = jax.lax.broadcasted_iota(jnp.int32, (H, h), 0)
    u_c = jax.lax.broadcasted_iota(jnp.int32, (H, h), 1)
    rrow = (u_r // _PATCH == u_c).astype(f32)
    pv = jnp.dot(rrow, g16, preferred_element_type=f32)
    c_r = jax.lax.broadcasted_iota(jnp.int32, (_LANE, W), 0)
    c_c = jax.lax.broadcasted_iota(jnp.int32, (_LANE, W), 1)
    rcol = (c_r == c_c // _PATCH).astype(f32)
    return jnp.dot(pv, rcol, preferred_element_type=f32)


def _fused_kernel(x4_ref,
                  w4, b_emb, ln1_g, ln1_b, qkv_w, qkv_b, proj_w, proj_b,
                  ln2_g, ln2_b, mlp1_w, mlp1_b, mlp2_w, mlp2_b,
                  w_xd, wimg, b_d, w2, b2,
                  out_ref, *, S, h, w, H, W, heads, dh, nimg):
    f32 = jnp.float32
    npix = _PATCH * _PATCH

    x4 = x4_ref[...].reshape(nimg, 4, h, _PATCH, w, _PATCH)
    p = x4.transpose(0, 2, 4, 1, 3, 5).reshape(nimg * S, 4 * npix)

    x = jnp.dot(p, w4[...], preferred_element_type=f32) + b_emb[...]

    hn = _ln(x, ln1_g[...], ln1_b[...])
    qkv = jnp.dot(_bf(hn), qkv_w[...], preferred_element_type=f32) + qkv_b[...]

    embed = hn.shape[-1]
    scale = 1.0 / (dh ** 0.5)
    attn_imgs = []
    for n in range(nimg):
        sl = slice(n * S, (n + 1) * S)
        head_outs = []
        for hd in range(heads):
            lo = hd * dh
            qh = qkv[sl, lo:lo + dh] * scale
            kh = qkv[sl, embed + lo:embed + lo + dh]
            vh = qkv[sl, 2 * embed + lo:2 * embed + lo + dh]
            s = jax.lax.dot_general(_bf(qh), _bf(kh), (((1,), (1,)), ((), ())),
                                    preferred_element_type=f32)
            m = jnp.max(s, axis=-1, keepdims=True)
            e = jnp.exp(s - m)
            e = e * pl.reciprocal(jnp.sum(e, axis=-1, keepdims=True),
                                  approx=True)
            head_outs.append(jnp.dot(_bf(e), _bf(vh),
                                     preferred_element_type=f32))
        attn_imgs.append(jnp.concatenate(head_outs, axis=-1))
    attn = jnp.concatenate(attn_imgs, axis=0)

    x = x + jnp.dot(_bf(attn), proj_w[...], preferred_element_type=f32) + proj_b[...]

    hn = _ln(x, ln2_g[...], ln2_b[...])
    mlp = jnp.dot(_bf(hn), mlp1_w[...], preferred_element_type=f32) + mlp1_b[...]
    mlp = jnp.maximum(mlp, 0.0)
    x = x + jnp.dot(_bf(mlp), mlp2_w[...], preferred_element_type=f32) + mlp2_b[...]

    d = jnp.dot(_bf(x), w_xd[...], preferred_element_type=f32)
    d = d + jnp.dot(p, wimg[...], preferred_element_type=f32)
    d = jnp.maximum(d + b_d[...], 0.0)
    a = jnp.dot(_bf(d), w2[...], preferred_element_type=f32) + b2[...]
    alpha = jax.nn.sigmoid(a[:, 0:1])

    for n in range(nimg):
        pred = _upsample(alpha[n * S:(n + 1) * S], h, w, H, W)
        out_ref[n] = pred.reshape(1, H, W)


def _full_spec(shape):
    nd = len(shape)
    return pl.BlockSpec(tuple(shape), lambda *_: (0,) * nd)


def kernel(patch_w, patch_b, guide_w, guide_b, ln1_g, ln1_b, qkv_w, qkv_b,
           proj_w, proj_b, ln2_g, ln2_b, mlp1_w, mlp1_b, mlp2_w, mlp2_b,
           neck_w, neck_b, maskhead_w, maskhead_b, m2m1_feas_w, m2m1_img_w,
           m2m1_mask_w, m2m1_b, m2m2_w, m2m2_b, image, guidance):
    f32 = jnp.float32
    B, Cin, H, W = image.shape
    h, w = H // _PATCH, W // _PATCH
    S = h * w
    embed = patch_w.shape[1]
    heads = 4
    dh = embed // heads
    npix = _PATCH * _PATCH
    nimg = 2

    x4 = jnp.concatenate([image, guidance], axis=1).astype(jnp.bfloat16)

    pw = patch_w.astype(f32).reshape(npix, Cin, embed)
    pw = pw.transpose(1, 0, 2).reshape(Cin * npix, embed)
    gw = jnp.broadcast_to(guide_w.astype(f32) / npix, (npix, embed))
    w4 = jnp.concatenate([pw, gw], axis=0).astype(jnp.bfloat16)
    b_emb = patch_b + guide_b

    w1f_eff = (m2m1_feas_w.astype(f32)
               + maskhead_w.astype(f32) @ m2m1_mask_w.astype(f32))
    w_xd = (neck_w.astype(f32) @ w1f_eff).astype(jnp.bfloat16)
    b_d = (m2m1_b + maskhead_b @ m2m1_mask_w.astype(f32)
           + neck_b @ w1f_eff)

    dec_h = m2m1_img_w.shape[1]
    wi = jnp.broadcast_to(m2m1_img_w.astype(f32)[:, None] / npix,
                          (Cin, npix, dec_h)).reshape(Cin * npix, dec_h)
    wimg = jnp.concatenate([wi, jnp.zeros((npix, dec_h), f32)], axis=0)
    wimg = wimg.astype(jnp.bfloat16)

    weights = [w4, b_emb, ln1_g, ln1_b, qkv_w, qkv_b, proj_w, proj_b,
               ln2_g, ln2_b, mlp1_w, mlp1_b, mlp2_w, mlp2_b,
               w_xd, wimg, b_d, m2m2_w, m2m2_b]
    w_specs = [_full_spec(x.shape) for x in weights]

    body = functools.partial(_fused_kernel, S=S, h=h, w=w, H=H, W=W,
                             heads=heads, dh=dh, nimg=nimg)
    pred = pl.pallas_call(
        body,
        out_shape=jax.ShapeDtypeStruct((B, 1, H, W), f32),
        grid=(B // nimg,),
        in_specs=[pl.BlockSpec((nimg, Cin + 1, H, W),
                               lambda b: (b, 0, 0, 0))] + w_specs,
        out_specs=pl.BlockSpec((nimg, 1, H, W), lambda b: (b, 0, 0, 0)),
        compiler_params=pltpu.CompilerParams(
            dimension_semantics=("parallel",)),
    )(x4, *weights)
    return pred


# speedup vs baseline: 1.3392x; 1.0294x over previous
import functools

import jax
import jax.numpy as jnp
from jax.experimental import pallas as pl
from jax.experimental.pallas import tpu as pltpu

_PATCH = 8


def _ln(x, g, b, eps=1e-6):
    mu = jnp.mean(x, axis=-1, keepdims=True)
    var = jnp.mean((x - mu) ** 2, axis=-1, keepdims=True)
    return (x - mu) * jax.lax.rsqrt(var + eps) * g + b


def _bf(x):
    return x.astype(jnp.bfloat16)


def _fused_kernel(z_ref,
                  xmask, xmaskf, rs, rc,
                  w_ext, b_emb, ln1_g, ln1_b, qkv_w, qkv_b, proj_w, proj_b,
                  ln2_g, ln2_b, mlp1_w, mlp1_b, mlp2_w, mlp2_b,
                  w_xd, b_d, w2, b2,
                  out_ref, *, S, h, w, H, W, heads, dh, nimg):
    f32 = jnp.float32
    M = nimg * S

    blk = z_ref[...]
    pieces = []
    for py in range(_PATCH):
        for c in range(4):
            a = blk[:, py, c]
            ar = jnp.broadcast_to(a[:, :, None, :], (nimg, h, w, W))
            pieces.append(ar.reshape(M, W))
    pext = jnp.concatenate(pieces, axis=1)
    npc = len(pieces)
    pext = (pext.reshape(M, npc, W) * xmask[...][:, None, :]).reshape(M, npc * W)

    embed = ln1_g.shape[-1]
    xe = jnp.dot(pext, w_ext[...], preferred_element_type=f32)
    x = xe[:, :embed] + b_emb[...]
    d_img = xe[:, embed:]

    hn = _ln(x, ln1_g[...], ln1_b[...])
    qkv = jnp.dot(_bf(hn), qkv_w[...], preferred_element_type=f32) + qkv_b[...]

    scale = 1.0 / (dh ** 0.5)
    attn_imgs = []
    for n in range(nimg):
        sl = slice(n * S, (n + 1) * S)
        head_outs = []
        for hd in range(heads):
            lo = hd * dh
            qh = qkv[sl, lo:lo + dh] * scale
            kh = qkv[sl, embed + lo:embed + lo + dh]
            vh = qkv[sl, 2 * embed + lo:2 * embed + lo + dh]
            s = jax.lax.dot_general(_bf(qh), _bf(kh), (((1,), (1,)), ((), ())),
                                    preferred_element_type=f32)
            m = jnp.max(s, axis=-1, keepdims=True)
            e = jnp.exp(s - m)
            e = e * pl.reciprocal(jnp.sum(e, axis=-1, keepdims=True),
                                  approx=True)
            head_outs.append(jnp.dot(_bf(e), _bf(vh),
                                     preferred_element_type=f32))
        attn_imgs.append(jnp.concatenate(head_outs, axis=-1))
    attn = jnp.concatenate(attn_imgs, axis=0)

    x = x + jnp.dot(_bf(attn), proj_w[...], preferred_element_type=f32) + proj_b[...]

    hn = _ln(x, ln2_g[...], ln2_b[...])
    mlp = jnp.dot(_bf(hn), mlp1_w[...], preferred_element_type=f32) + mlp1_b[...]
    mlp = jnp.maximum(mlp, 0.0)
    x = x + jnp.dot(_bf(mlp), mlp2_w[...], preferred_element_type=f32) + mlp2_b[...]

    d = jnp.dot(_bf(x), w_xd[...], preferred_element_type=f32) + d_img
    d = jnp.maximum(d + b_d[...], 0.0)
    a = jnp.dot(_bf(d), w2[...], preferred_element_type=f32) + b2[...]
    alpha = jax.nn.sigmoid(a[:, 0:1])

    av = jnp.broadcast_to(alpha, (M, W))
    asel = av * xmaskf[...]
    for n in range(nimg):
        pv = jnp.dot(rs[...], asel[n * S:(n + 1) * S],
                     preferred_element_type=f32)
        pred = jnp.dot(pv, rc[...], preferred_element_type=f32)
        out_ref[n] = pred.reshape(1, H, W)


def _full_spec(shape):
    nd = len(shape)
    return pl.BlockSpec(tuple(shape), lambda *_: (0,) * nd)


def kernel(patch_w, patch_b, guide_w, guide_b, ln1_g, ln1_b, qkv_w, qkv_b,
           proj_w, proj_b, ln2_g, ln2_b, mlp1_w, mlp1_b, mlp2_w, mlp2_b,
           neck_w, neck_b, maskhead_w, maskhead_b, m2m1_feas_w, m2m1_img_w,
           m2m1_mask_w, m2m1_b, m2m2_w, m2m2_b, image, guidance):
    f32 = jnp.float32
    B, Cin, H, W = image.shape
    h, w = H // _PATCH, W // _PATCH
    S = h * w
    embed = patch_w.shape[1]
    heads = 4
    dh = embed // heads
    npix = _PATCH * _PATCH
    nimg = 4
    M = nimg * S

    x4 = jnp.concatenate([image, guidance], axis=1).astype(jnp.bfloat16)
    z = x4.reshape(B, Cin + 1, h, _PATCH, W).transpose(0, 3, 1, 2, 4)

    pw = patch_w.astype(f32).reshape(npix, Cin, embed)
    pw = pw.transpose(1, 0, 2).reshape(Cin * npix, embed)
    gw = jnp.broadcast_to(guide_w.astype(f32) / npix, (npix, embed))
    w4 = jnp.concatenate([pw, gw], axis=0)
    b_emb = patch_b + guide_b

    w1f_eff = (m2m1_feas_w.astype(f32)
               + maskhead_w.astype(f32) @ m2m1_mask_w.astype(f32))
    w_xd = (neck_w.astype(f32) @ w1f_eff).astype(jnp.bfloat16)
    b_d = (m2m1_b + maskhead_b @ m2m1_mask_w.astype(f32)
           + neck_b @ w1f_eff)

    dec_h = m2m1_img_w.shape[1]
    wi = jnp.broadcast_to(m2m1_img_w.astype(f32)[:, None] / npix,
                          (Cin, npix, dec_h)).reshape(Cin * npix, dec_h)
    wimg = jnp.concatenate([wi, jnp.zeros((npix, dec_h), f32)], axis=0)

    def _ext(wmat):
        wr = wmat.reshape(Cin + 1, npix // _PATCH, _PATCH, -1)
        wr = wr.transpose(1, 0, 2, 3)
        wr = jnp.broadcast_to(wr[:, :, None, :, :],
                              (npix // _PATCH, Cin + 1, w, _PATCH, wr.shape[-1]))
        return wr.reshape((Cin + 1) * npix * w, wr.shape[-1])

    w_ext = jnp.concatenate([_ext(w4), _ext(wimg)], axis=1).astype(jnp.bfloat16)

    t_i = jax.lax.broadcasted_iota(jnp.int32, (M, W), 0)
    l_i = jax.lax.broadcasted_iota(jnp.int32, (M, W), 1)
    xmask = (l_i // _PATCH == t_i % w).astype(jnp.bfloat16)
    u_r = jax.lax.broadcasted_iota(jnp.int32, (H, S), 0)
    u_t = jax.lax.broadcasted_iota(jnp.int32, (H, S), 1)
    rs = (u_t // w == u_r // _PATCH).astype(f32)
    c_r = jax.lax.broadcasted_iota(jnp.int32, (W, W), 0)
    c_c = jax.lax.broadcasted_iota(jnp.int32, (W, W), 1)
    rc = (c_r // _PATCH == c_c // _PATCH).astype(f32) * (
        (c_r % _PATCH) == 0).astype(f32)

    weights = [w_ext, b_emb, ln1_g, ln1_b, qkv_w, qkv_b, proj_w, proj_b,
               ln2_g, ln2_b, mlp1_w, mlp1_b, mlp2_w, mlp2_b,
               w_xd, b_d, m2m2_w, m2m2_b]
    w_specs = [_full_spec(x.shape) for x in weights]

    body = functools.partial(_fused_kernel, S=S, h=h, w=w, H=H, W=W,
                             heads=heads, dh=dh, nimg=nimg)
    pred = pl.pallas_call(
        body,
        out_shape=jax.ShapeDtypeStruct((B, 1, H, W), f32),
        grid=(B // nimg,),
        in_specs=[pl.BlockSpec((nimg, _PATCH, Cin + 1, h, W),
                               lambda b: (b, 0, 0, 0, 0)),
                  _full_spec((M, W)), _full_spec((M, W)),
                  _full_spec((H, S)), _full_spec((W, W))]
                 + w_specs,
        out_specs=pl.BlockSpec((nimg, 1, H, W), lambda b: (b, 0, 0, 0)),
        compiler_params=pltpu.CompilerParams(
            dimension_semantics=("parallel",)),
    )(z, xmask, xmask.astype(f32), rs, rc, *weights)
    return pred


# speedup vs baseline: 1.3443x; 1.0039x over previous
import functools

import jax
import jax.numpy as jnp
from jax.experimental import pallas as pl
from jax.experimental.pallas import tpu as pltpu

_PATCH = 8


def _ln(x, g, b, eps=1e-6):
    mu = jnp.mean(x, axis=-1, keepdims=True)
    var = jnp.mean((x - mu) ** 2, axis=-1, keepdims=True)
    return (x - mu) * jax.lax.rsqrt(var + eps) * g + b


def _bf(x):
    return x.astype(jnp.bfloat16)


def _fused_kernel(z_ref,
                  xmask, xmaskf, rs, rc,
                  w_ext, b_emb, ln1_g, ln1_b, qkv_w, qkv_b, proj_w, proj_b,
                  ln2_g, ln2_b, mlp1_w, mlp1_b, mlp2_w, mlp2_b,
                  w_xd, b_d, w2, b2,
                  out_ref, *, S, h, w, H, W, heads, dh, nimg):
    f32 = jnp.float32
    M = nimg * S

    blk = z_ref[...]
    xm = xmask[...]
    w_ext_v = w_ext[...]
    nchunk = 4
    pc_per = (_PATCH * 4) // nchunk
    xe = None
    for ck in range(nchunk):
        pieces = []
        for pi in range(pc_per):
            flat = ck * pc_per + pi
            py, c = flat // 4, flat % 4
            a = blk[:, py, c]
            ar = jnp.broadcast_to(a[:, :, None, :], (nimg, h, w, W))
            pieces.append(ar.reshape(M, W))
        pck = jnp.concatenate(pieces, axis=1)
        pck = (pck.reshape(M, pc_per, W) * xm[:, None, :]).reshape(M, pc_per * W)
        part = jnp.dot(pck, w_ext_v[ck * pc_per * W:(ck + 1) * pc_per * W, :],
                       preferred_element_type=f32)
        xe = part if xe is None else xe + part

    embed = ln1_g.shape[-1]
    x = xe[:, :embed] + b_emb[...]
    d_img = xe[:, embed:]

    hn = _ln(x, ln1_g[...], ln1_b[...])
    qkv = jnp.dot(_bf(hn), qkv_w[...], preferred_element_type=f32) + qkv_b[...]

    scale = 1.0 / (dh ** 0.5)
    attn_imgs = []
    for n in range(nimg):
        sl = slice(n * S, (n + 1) * S)
        head_outs = []
        for hd in range(heads):
            lo = hd * dh
            qh = qkv[sl, lo:lo + dh] * scale
            kh = qkv[sl, embed + lo:embed + lo + dh]
            vh = qkv[sl, 2 * embed + lo:2 * embed + lo + dh]
            s = jax.lax.dot_general(_bf(qh), _bf(kh), (((1,), (1,)), ((), ())),
                                    preferred_element_type=f32)
            m = jnp.max(s, axis=-1, keepdims=True)
            e = jnp.exp(s - m)
            e = e * pl.reciprocal(jnp.sum(e, axis=-1, keepdims=True),
                                  approx=True)
            head_outs.append(jnp.dot(_bf(e), _bf(vh),
                                     preferred_element_type=f32))
        attn_imgs.append(jnp.concatenate(head_outs, axis=-1))
    attn = jnp.concatenate(attn_imgs, axis=0)

    x = x + jnp.dot(_bf(attn), proj_w[...], preferred_element_type=f32) + proj_b[...]

    hn = _ln(x, ln2_g[...], ln2_b[...])
    mlp = jnp.dot(_bf(hn), mlp1_w[...], preferred_element_type=f32) + mlp1_b[...]
    mlp = jnp.maximum(mlp, 0.0)
    x = x + jnp.dot(_bf(mlp), mlp2_w[...], preferred_element_type=f32) + mlp2_b[...]

    d = jnp.dot(_bf(x), w_xd[...], preferred_element_type=f32) + d_img
    d = jnp.maximum(d + b_d[...], 0.0)
    a = jnp.dot(_bf(d), w2[...], preferred_element_type=f32) + b2[...]
    alpha = jax.nn.sigmoid(a[:, 0:1])

    av = jnp.broadcast_to(alpha, (M, W))
    asel = av * xmaskf[...]
    for n in range(nimg):
        pv = jnp.dot(rs[...], asel[n * S:(n + 1) * S],
                     preferred_element_type=f32)
        pred = jnp.dot(pv, rc[...], preferred_element_type=f32)
        out_ref[n] = pred.reshape(1, H, W)


def _full_spec(shape):
    nd = len(shape)
    return pl.BlockSpec(tuple(shape), lambda *_: (0,) * nd)


def kernel(patch_w, patch_b, guide_w, guide_b, ln1_g, ln1_b, qkv_w, qkv_b,
           proj_w, proj_b, ln2_g, ln2_b, mlp1_w, mlp1_b, mlp2_w, mlp2_b,
           neck_w, neck_b, maskhead_w, maskhead_b, m2m1_feas_w, m2m1_img_w,
           m2m1_mask_w, m2m1_b, m2m2_w, m2m2_b, image, guidance):
    f32 = jnp.float32
    B, Cin, H, W = image.shape
    h, w = H // _PATCH, W // _PATCH
    S = h * w
    embed = patch_w.shape[1]
    heads = 4
    dh = embed // heads
    npix = _PATCH * _PATCH
    nimg = 4
    M = nimg * S
    K = (Cin + 1) * npix * w

    x4 = jnp.concatenate([image, guidance], axis=1).astype(jnp.bfloat16)
    z = x4.reshape(B, Cin + 1, h, _PATCH, W).transpose(0, 3, 1, 2, 4)

    pw = patch_w.astype(f32).reshape(npix, Cin, embed)
    pw = pw.transpose(1, 0, 2).reshape(Cin * npix, embed)
    gw = jnp.broadcast_to(guide_w.astype(f32) / npix, (npix, embed))
    w4 = jnp.concatenate([pw, gw], axis=0)
    b_emb = patch_b + guide_b

    w1f_eff = (m2m1_feas_w.astype(f32)
               + maskhead_w.astype(f32) @ m2m1_mask_w.astype(f32))
    w_xd = (neck_w.astype(f32) @ w1f_eff).astype(jnp.bfloat16)
    b_d = (m2m1_b + maskhead_b @ m2m1_mask_w.astype(f32)
           + neck_b @ w1f_eff)

    dec_h = m2m1_img_w.shape[1]
    wi = jnp.broadcast_to(m2m1_img_w.astype(f32)[:, None] / npix,
                          (Cin, npix, dec_h)).reshape(Cin * npix, dec_h)
    wimg = jnp.concatenate([wi, jnp.zeros((npix, dec_h), f32)], axis=0)

    def _ext(wmat):
        wr = wmat.reshape(Cin + 1, npix // _PATCH, _PATCH, -1)
        wr = wr.transpose(1, 0, 2, 3)
        wr = jnp.broadcast_to(wr[:, :, None, :, :],
                              (npix // _PATCH, Cin + 1, w, _PATCH, wr.shape[-1]))
        return wr.reshape(K, wr.shape[-1])

    w_ext = jnp.concatenate([_ext(w4), _ext(wimg)], axis=1).astype(jnp.bfloat16)

    t_i = jax.lax.broadcasted_iota(jnp.int32, (M, W), 0)
    l_i = jax.lax.broadcasted_iota(jnp.int32, (M, W), 1)
    xmask = (l_i // _PATCH == t_i % w).astype(jnp.bfloat16)
    u_r = jax.lax.broadcasted_iota(jnp.int32, (H, S), 0)
    u_t = jax.lax.broadcasted_iota(jnp.int32, (H, S), 1)
    rs = (u_t // w == u_r // _PATCH).astype(f32)
    c_r = jax.lax.broadcasted_iota(jnp.int32, (W, W), 0)
    c_c = jax.lax.broadcasted_iota(jnp.int32, (W, W), 1)
    rc = (c_r // _PATCH == c_c // _PATCH).astype(f32) * (
        (c_r % _PATCH) == 0).astype(f32)

    weights = [w_ext, b_emb, ln1_g, ln1_b, qkv_w, qkv_b, proj_w, proj_b,
               ln2_g, ln2_b, mlp1_w, mlp1_b, mlp2_w, mlp2_b,
               w_xd, b_d, m2m2_w, m2m2_b]
    w_specs = [_full_spec(x.shape) for x in weights]

    body = functools.partial(_fused_kernel, S=S, h=h, w=w, H=H, W=W,
                             heads=heads, dh=dh, nimg=nimg)
    pred = pl.pallas_call(
        body,
        out_shape=jax.ShapeDtypeStruct((B, 1, H, W), f32),
        grid=(B // nimg,),
        in_specs=[pl.BlockSpec((nimg, _PATCH, Cin + 1, h, W),
                               lambda b: (b, 0, 0, 0, 0)),
                  _full_spec((M, W)), _full_spec((M, W)),
                  _full_spec((H, S)), _full_spec((W, W))]
                 + w_specs,
        out_specs=pl.BlockSpec((nimg, 1, H, W), lambda b: (b, 0, 0, 0)),
        compiler_params=pltpu.CompilerParams(
            dimension_semantics=("parallel",)),
    )(z, xmask, xmask.astype(f32), rs, rc, *weights)
    return pred
